# baseline (speedup 1.0000x reference)
import jax
import jax.numpy as jnp
from jax.experimental import pallas as pl

_EPS = 1e-6
_H_BLOCK = 8
_WINDOW = 2048


def _vq_rot_kernel(ze_ref, eb_ref, emb_ref, a2_ref, b2_ref,
                   zq_ref, idx_ref, acc_ref):
    i = pl.program_id(0)
    zblk = ze_ref[...]
    eb = eb_ref[...]
    emb = emb_ref[...]
    a2 = a2_ref[...]
    b2 = b2_ref[...]
    _, d, hb, w = zblk.shape
    t = hb * w
    c = emb.shape[0]

    z = jnp.transpose(zblk[0], (1, 2, 0)).reshape(t, d)
    zb = z.astype(jnp.bfloat16)

    ab = jax.lax.dot_general(zb, eb, (((1,), (1,)), ((), ())),
                             preferred_element_type=jnp.float32)
    dist = (a2 - 2.0 * ab) + b2

    carry_v = jnp.full((t, 1), jnp.inf, jnp.float32)
    carry_i = jnp.zeros((t, 1), jnp.int32)
    iota_w = jax.lax.broadcasted_iota(jnp.int32, (t, _WINDOW), 1)
    for wi in range(c // _WINDOW):
        dw = jax.lax.slice(dist, (0, wi * _WINDOW), (t, (wi + 1) * _WINDOW))
        m = jnp.min(dw, axis=1, keepdims=True)
        mi = jnp.min(jnp.where(dw == m, iota_w, _WINDOW), axis=1,
                     keepdims=True) + wi * _WINDOW
        take = m < carry_v
        carry_v = jnp.where(take, m.astype(jnp.bfloat16).astype(jnp.float32),
                            carry_v)
        carry_i = jnp.where(take, mi, carry_i)
    idx_ref[...] = carry_i

    iota = jax.lax.broadcasted_iota(jnp.int32, (t, c), 1)
    onehot = (iota == carry_i).astype(jnp.bfloat16)
    e_lo = (emb - eb.astype(jnp.float32)).astype(jnp.bfloat16)
    q_hi = jax.lax.dot_general(onehot, eb, (((1,), (0,)), ((), ())),
                               preferred_element_type=jnp.float32)
    q_lo = jax.lax.dot_general(onehot, e_lo, (((1,), (0,)), ((), ())),
                               preferred_element_type=jnp.float32)
    q = q_hi + q_lo

    z_norm = jnp.sqrt(jnp.sum(z * z, axis=1, keepdims=True))
    q_norm = jnp.sqrt(jnp.sum(q * q, axis=1, keepdims=True))
    z_hat = z / (z_norm + _EPS)
    q_hat = q / (q_norm + _EPS)
    v = z_hat - q_hat
    v = v / (jnp.sqrt(jnp.sum(v * v, axis=1, keepdims=True)) + _EPS)
    rz = z - 2.0 * jnp.sum(v * z, axis=1, keepdims=True) * v
    s = q_norm / (z_norm + _EPS)
    q_tilde = s * rz

    zq_ref[...] = jnp.transpose(q_tilde.reshape(hb, w, d), (2, 0, 1))[None]

    diff = z - q
    part = jnp.sum(diff * diff).reshape(1, 1)

    @pl.when(i == 0)
    def _():
        acc_ref[...] = jnp.zeros((1, 1), jnp.float32)

    acc_ref[...] += part


@jax.jit
def kernel(z_e, embedding):
    b, d, h, w = z_e.shape
    c = embedding.shape[0]
    n = b * h * w
    hb = _H_BLOCK
    t = hb * w
    blocks_per_b = h // hb
    e_bf = embedding.astype(jnp.bfloat16)
    a2 = jnp.sum(jnp.transpose(z_e, (0, 2, 3, 1)).reshape(n, d) ** 2,
                 axis=1, keepdims=True)
    b2 = jnp.sum(embedding ** 2, axis=1).reshape(1, c)

    z_q, idx, acc = pl.pallas_call(
        _vq_rot_kernel,
        grid=(n // t,),
        in_specs=[
            pl.BlockSpec((1, d, hb, w),
                         lambda i: (i // blocks_per_b, 0, i % blocks_per_b, 0)),
            pl.BlockSpec((c, d), lambda i: (0, 0)),
            pl.BlockSpec((c, d), lambda i: (0, 0)),
            pl.BlockSpec((t, 1), lambda i: (i, 0)),
            pl.BlockSpec((1, c), lambda i: (0, 0)),
        ],
        out_specs=[
            pl.BlockSpec((1, d, hb, w),
                         lambda i: (i // blocks_per_b, 0, i % blocks_per_b, 0)),
            pl.BlockSpec((t, 1), lambda i: (i, 0)),
            pl.BlockSpec((1, 1), lambda i: (0, 0)),
        ],
        out_shape=[
            jax.ShapeDtypeStruct((b, d, h, w), jnp.float32),
            jax.ShapeDtypeStruct((n, 1), jnp.int32),
            jax.ShapeDtypeStruct((1, 1), jnp.float32),
        ],
    )(z_e, e_bf, embedding, a2, b2)

    indices_out = idx.reshape(b, h, w)
    commit_loss = (0.25 / (n * d)) * acc[0, 0]
    return (z_q, indices_out, commit_loss)


# speedup vs baseline: 1.3427x; 1.3427x over previous
import functools

import jax
import jax.numpy as jnp
from jax import lax
from jax.experimental import pallas as pl
from jax.experimental.pallas import tpu as pltpu
from jax.experimental.pallas import tpu_sc as plsc

_EPS = 1e-6
_TOKEN_BLOCK = 256
_ROT_BLOCK = 1024
_WINDOW = 2048


def _argmin_kernel(zb_ref, eb_ref, a2_ref, b2_ref, idx_ref):
    zb = zb_ref[...]
    eb = eb_ref[...]
    a2 = a2_ref[...]
    b2 = b2_ref[...]
    t = zb.shape[0]
    c = eb.shape[0]

    ab = jax.lax.dot_general(zb, eb, (((1,), (1,)), ((), ())),
                             preferred_element_type=jnp.float32)
    dist = (a2 - 2.0 * ab) + b2

    carry_v = jnp.full((t, 1), jnp.inf, jnp.float32)
    carry_i = jnp.zeros((t, 1), jnp.int32)
    iota_w = jax.lax.broadcasted_iota(jnp.int32, (t, _WINDOW), 1)
    for wi in range(c // _WINDOW):
        dw = jax.lax.slice(dist, (0, wi * _WINDOW), (t, (wi + 1) * _WINDOW))
        m = jnp.min(dw, axis=1, keepdims=True)
        mi = jnp.min(jnp.where(dw == m, iota_w, _WINDOW), axis=1,
                     keepdims=True) + wi * _WINDOW
        take = m < carry_v
        carry_v = jnp.where(take, m.astype(jnp.bfloat16).astype(jnp.float32),
                            carry_v)
        carry_i = jnp.where(take, mi, carry_i)
    idx_ref[...] = carry_i


def _make_sc_gather(n, c, d):
    info = plsc.get_sparse_core_info()
    nw = info.num_cores * info.num_subcores
    b_per_w = n // nw
    mesh = plsc.VectorSubcoreMesh(core_axis_name="c", subcore_axis_name="s")

    @functools.partial(
        pl.kernel, mesh=mesh,
        out_type=jax.ShapeDtypeStruct((n, d), jnp.float32),
        scratch_types=[
            pltpu.VMEM((b_per_w,), jnp.int32),
            pltpu.VMEM((b_per_w, d), jnp.float32),
            pltpu.SemaphoreType.DMA,
        ],
    )
    def gather(table_hbm, idx_hbm, out_hbm, idx_v, rows_v, sem):
        wid = lax.axis_index("s") * info.num_cores + lax.axis_index("c")
        base = wid * b_per_w
        pltpu.sync_copy(idx_hbm.at[pl.ds(base, b_per_w)], idx_v)
        pltpu.async_copy(table_hbm.at[idx_v], rows_v, sem).wait()
        pltpu.sync_copy(rows_v, out_hbm.at[pl.ds(base, b_per_w)])

    return gather


def _rotation_kernel(z_ref, q_ref, zq_ref, acc_ref):
    i = pl.program_id(0)
    z = z_ref[...]
    d = z.shape[1]
    q = q_ref[:, :d]

    z_norm = jnp.sqrt(jnp.sum(z * z, axis=1, keepdims=True))
    q_norm = jnp.sqrt(jnp.sum(q * q, axis=1, keepdims=True))
    z_hat = z / (z_norm + _EPS)
    q_hat = q / (q_norm + _EPS)
    v = z_hat - q_hat
    v = v / (jnp.sqrt(jnp.sum(v * v, axis=1, keepdims=True)) + _EPS)
    rz = z - 2.0 * jnp.sum(v * z, axis=1, keepdims=True) * v
    s = q_norm / (z_norm + _EPS)
    zq_ref[...] = s * rz

    diff = z - q
    part = jnp.sum(diff * diff).reshape(1, 1)

    @pl.when(i == 0)
    def _():
        acc_ref[...] = jnp.zeros((1, 1), jnp.float32)

    acc_ref[...] += part


@jax.jit
def kernel(z_e, embedding):
    b, d, h, w = z_e.shape
    c = embedding.shape[0]
    n = b * h * w
    t = _TOKEN_BLOCK
    z_flat = jnp.transpose(z_e, (0, 2, 3, 1)).reshape(n, d)
    z_bf = z_flat.astype(jnp.bfloat16)
    e_bf = embedding.astype(jnp.bfloat16)
    a2 = jnp.sum(z_flat ** 2, axis=1, keepdims=True)
    b2 = jnp.sum(embedding ** 2, axis=1).reshape(1, c)

    idx = pl.pallas_call(
        _argmin_kernel,
        grid=(n // t,),
        in_specs=[
            pl.BlockSpec((t, d), lambda i: (i, 0)),
            pl.BlockSpec((c, d), lambda i: (0, 0)),
            pl.BlockSpec((t, 1), lambda i: (i, 0)),
            pl.BlockSpec((1, c), lambda i: (0, 0)),
        ],
        out_specs=pl.BlockSpec((t, 1), lambda i: (i, 0)),
        out_shape=jax.ShapeDtypeStruct((n, 1), jnp.int32),
    )(z_bf, e_bf, a2, b2)

    emb_pad = jnp.pad(embedding, ((0, 0), (0, 128 - d)))
    q_flat = _make_sc_gather(n, c, 128)(emb_pad, idx.reshape(n))

    rt = _ROT_BLOCK
    q_tilde, acc = pl.pallas_call(
        _rotation_kernel,
        grid=(n // rt,),
        in_specs=[
            pl.BlockSpec((rt, d), lambda i: (i, 0)),
            pl.BlockSpec((rt, 128), lambda i: (i, 0)),
        ],
        out_specs=[
            pl.BlockSpec((rt, d), lambda i: (i, 0)),
            pl.BlockSpec((1, 1), lambda i: (0, 0)),
        ],
        out_shape=[
            jax.ShapeDtypeStruct((n, d), jnp.float32),
            jax.ShapeDtypeStruct((1, 1), jnp.float32),
        ],
    )(z_flat, q_flat)

    z_q = jnp.transpose(q_tilde.reshape(b, h, w, d), (0, 3, 1, 2))
    indices_out = idx.reshape(b, h, w)
    commit_loss = (0.25 / (n * d)) * acc[0, 0]
    return (z_q, indices_out, commit_loss)


# speedup vs baseline: 1.4082x; 1.0488x over previous
import functools

import jax
import jax.numpy as jnp
from jax import lax
from jax.experimental import pallas as pl
from jax.experimental.pallas import tpu as pltpu
from jax.experimental.pallas import tpu_sc as plsc

_EPS = 1e-6
_TOKEN_BLOCK = 512
_ROT_BLOCK = 2048
_WINDOW = 2048


def _argmin_kernel(zb_ref, eb_ref, a2_ref, b2_ref, idx_ref):
    zb = zb_ref[...]
    eb = eb_ref[...]
    a2 = a2_ref[...]
    b2 = b2_ref[...]
    t = zb.shape[0]
    c = eb.shape[0]

    ab = jax.lax.dot_general(zb, eb, (((1,), (1,)), ((), ())),
                             preferred_element_type=jnp.float32)
    dist = (a2 - 2.0 * ab) + b2

    carry_v = jnp.full((t, 1), jnp.inf, jnp.float32)
    carry_i = jnp.zeros((t, 1), jnp.int32)
    iota_w = jax.lax.broadcasted_iota(jnp.int32, (t, _WINDOW), 1)
    for wi in range(c // _WINDOW):
        dw = jax.lax.slice(dist, (0, wi * _WINDOW), (t, (wi + 1) * _WINDOW))
        m = jnp.min(dw, axis=1, keepdims=True)
        mi = jnp.min(jnp.where(dw == m, iota_w, _WINDOW), axis=1,
                     keepdims=True) + wi * _WINDOW
        take = m < carry_v
        carry_v = jnp.where(take, m.astype(jnp.bfloat16).astype(jnp.float32),
                            carry_v)
        carry_i = jnp.where(take, mi, carry_i)
    idx_ref[...] = carry_i


def _make_sc_gather(n, c, d):
    info = plsc.get_sparse_core_info()
    nw = info.num_cores * info.num_subcores
    b_per_w = n // nw
    mesh = plsc.VectorSubcoreMesh(core_axis_name="c", subcore_axis_name="s")

    @functools.partial(
        pl.kernel, mesh=mesh,
        out_type=jax.ShapeDtypeStruct((n, d), jnp.float32),
        scratch_types=[
            pltpu.VMEM((b_per_w,), jnp.int32),
            pltpu.VMEM((b_per_w, d), jnp.float32),
            pltpu.SemaphoreType.DMA,
        ],
    )
    def gather(table_hbm, idx_hbm, out_hbm, idx_v, rows_v, sem):
        wid = lax.axis_index("s") * info.num_cores + lax.axis_index("c")
        base = wid * b_per_w
        pltpu.sync_copy(idx_hbm.at[pl.ds(base, b_per_w)], idx_v)
        pltpu.async_copy(table_hbm.at[idx_v], rows_v, sem).wait()
        pltpu.sync_copy(rows_v, out_hbm.at[pl.ds(base, b_per_w)])

    return gather


def _rotation_kernel(z_ref, q_ref, zq_ref, acc_ref):
    i = pl.program_id(0)
    z = z_ref[...]
    d = z.shape[1]
    q = q_ref[:, :d]

    z_norm = jnp.sqrt(jnp.sum(z * z, axis=1, keepdims=True))
    q_norm = jnp.sqrt(jnp.sum(q * q, axis=1, keepdims=True))
    z_hat = z / (z_norm + _EPS)
    q_hat = q / (q_norm + _EPS)
    v = z_hat - q_hat
    v = v / (jnp.sqrt(jnp.sum(v * v, axis=1, keepdims=True)) + _EPS)
    rz = z - 2.0 * jnp.sum(v * z, axis=1, keepdims=True) * v
    s = q_norm / (z_norm + _EPS)
    zq_ref[...] = s * rz

    diff = z - q
    part = jnp.sum(diff * diff).reshape(1, 1)

    @pl.when(i == 0)
    def _():
        acc_ref[...] = jnp.zeros((1, 1), jnp.float32)

    acc_ref[...] += part


@jax.jit
def kernel(z_e, embedding):
    b, d, h, w = z_e.shape
    c = embedding.shape[0]
    n = b * h * w
    t = _TOKEN_BLOCK
    z_flat = jnp.transpose(z_e, (0, 2, 3, 1)).reshape(n, d)
    z_bf = z_flat.astype(jnp.bfloat16)
    e_bf = embedding.astype(jnp.bfloat16)
    a2 = jnp.sum(z_flat ** 2, axis=1, keepdims=True)
    b2 = jnp.sum(embedding ** 2, axis=1).reshape(1, c)

    idx = pl.pallas_call(
        _argmin_kernel,
        grid=(n // t,),
        in_specs=[
            pl.BlockSpec((t, d), lambda i: (i, 0)),
            pl.BlockSpec((c, d), lambda i: (0, 0)),
            pl.BlockSpec((t, 1), lambda i: (i, 0)),
            pl.BlockSpec((1, c), lambda i: (0, 0)),
        ],
        out_specs=pl.BlockSpec((t, 1), lambda i: (i, 0)),
        out_shape=jax.ShapeDtypeStruct((n, 1), jnp.int32),
    )(z_bf, e_bf, a2, b2)

    emb_pad = jnp.pad(embedding, ((0, 0), (0, 128 - d)))
    q_flat = _make_sc_gather(n, c, 128)(emb_pad, idx.reshape(n))

    rt = _ROT_BLOCK
    q_tilde, acc = pl.pallas_call(
        _rotation_kernel,
        grid=(n // rt,),
        in_specs=[
            pl.BlockSpec((rt, d), lambda i: (i, 0)),
            pl.BlockSpec((rt, 128), lambda i: (i, 0)),
        ],
        out_specs=[
            pl.BlockSpec((rt, d), lambda i: (i, 0)),
            pl.BlockSpec((1, 1), lambda i: (0, 0)),
        ],
        out_shape=[
            jax.ShapeDtypeStruct((n, d), jnp.float32),
            jax.ShapeDtypeStruct((1, 1), jnp.float32),
        ],
    )(z_flat, q_flat)

    z_q = jnp.transpose(q_tilde.reshape(b, h, w, d), (0, 3, 1, 2))
    indices_out = idx.reshape(b, h, w)
    commit_loss = (0.25 / (n * d)) * acc[0, 0]
    return (z_q, indices_out, commit_loss)
